# Initial kernel scaffold; baseline (speedup 1.0000x reference)
import jax
import jax.numpy as jnp
from jax.experimental import pallas as pl


def kernel(features, scores, targets, flags, lut_b1, lut_b2, lut_b3, lut_b4):
    raise NotImplementedError("write your pallas kernel here")


# speedup vs baseline: 1.9581x; 1.9581x over previous
import functools

import jax
import jax.numpy as jnp
from jax.experimental import pallas as pl
from jax.experimental.pallas import tpu as pltpu

_NUM_CLASSES = 100000
_C_BLK = 4096


def _oim_body(f_ref, t_ref, l1_ref, l2_ref, l3_ref, l4_ref,
              logits_ref, loss_ref, m_ref, s_ref, tl_ref,
              *, nblk, nclasses, cblk):
    i = pl.program_id(0)

    @pl.when(i == 0)
    def _init():
        m_ref[...] = jnp.full_like(m_ref, -jnp.inf)
        s_ref[...] = jnp.zeros_like(s_ref)
        tl_ref[...] = jnp.zeros_like(tl_ref)

    f = f_ref[...]
    dn = (((1,), (1,)), ((), ()))
    acc = jax.lax.dot_general(f[:, 0, :], l1_ref[...], dn,
                              preferred_element_type=jnp.float32)
    acc += jax.lax.dot_general(f[:, 1, :], l2_ref[...], dn,
                               preferred_element_type=jnp.float32)
    acc += jax.lax.dot_general(f[:, 2, :], l3_ref[...], dn,
                               preferred_element_type=jnp.float32)
    acc += jax.lax.dot_general(f[:, 3, :], l4_ref[...], dn,
                               preferred_element_type=jnp.float32)
    logits_ref[...] = acc

    col = jax.lax.broadcasted_iota(jnp.int32, acc.shape, 1) + i * cblk
    valid = col < nclasses
    masked = jnp.where(valid, acc, -jnp.inf)
    bmax = jnp.max(masked, axis=1, keepdims=True)
    m_old = m_ref[...]
    m_new = jnp.maximum(m_old, bmax)
    p = jnp.where(valid, jnp.exp(acc - m_new), 0.0)
    s_ref[...] = s_ref[...] * jnp.exp(m_old - m_new) + jnp.sum(
        p, axis=1, keepdims=True)
    m_ref[...] = m_new

    t = t_ref[...]
    tl_ref[...] += jnp.sum(jnp.where(col == t, acc, 0.0), axis=1,
                           keepdims=True)

    @pl.when(i == nblk - 1)
    def _fin():
        lse = m_ref[...] + jnp.log(s_ref[...])
        loss_ref[...] = jnp.mean(lse - tl_ref[...]).reshape(1, 1)


def kernel(features, scores, targets, flags, lut_b1, lut_b2, lut_b3,
           lut_b4):
    batch, _, nfeat = features.shape
    nclasses = lut_b1.shape[0]
    nblk = pl.cdiv(nclasses, _C_BLK)
    t2 = targets.astype(jnp.int32).reshape(batch, 1)

    body = functools.partial(_oim_body, nblk=nblk, nclasses=nclasses,
                             cblk=_C_BLK)
    logits, loss = pl.pallas_call(
        body,
        grid=(nblk,),
        in_specs=[
            pl.BlockSpec((batch, 4, nfeat), lambda i: (0, 0, 0)),
            pl.BlockSpec((batch, 1), lambda i: (0, 0)),
            pl.BlockSpec((_C_BLK, nfeat), lambda i: (i, 0)),
            pl.BlockSpec((_C_BLK, nfeat), lambda i: (i, 0)),
            pl.BlockSpec((_C_BLK, nfeat), lambda i: (i, 0)),
            pl.BlockSpec((_C_BLK, nfeat), lambda i: (i, 0)),
        ],
        out_specs=[
            pl.BlockSpec((batch, _C_BLK), lambda i: (0, i)),
            pl.BlockSpec((1, 1), lambda i: (0, 0)),
        ],
        out_shape=[
            jax.ShapeDtypeStruct((batch, nclasses), jnp.float32),
            jax.ShapeDtypeStruct((1, 1), jnp.float32),
        ],
        scratch_shapes=[
            pltpu.VMEM((batch, 1), jnp.float32),
            pltpu.VMEM((batch, 1), jnp.float32),
            pltpu.VMEM((batch, 1), jnp.float32),
        ],
        compiler_params=pltpu.CompilerParams(
            dimension_semantics=("arbitrary",)),
    )(features, t2, lut_b1, lut_b2, lut_b3, lut_b4)
    return (loss[0, 0], logits)


# speedup vs baseline: 2.0437x; 1.0437x over previous
import functools

import jax
import jax.numpy as jnp
from jax.experimental import pallas as pl
from jax.experimental.pallas import tpu as pltpu

_NUM_CLASSES = 100000
_C_BLK = 8192


def _oim_body(f_ref, t_ref, l1_ref, l2_ref, l3_ref, l4_ref,
              logits_ref, loss_ref, m_ref, s_ref, tl_ref,
              *, nblk, nclasses, cblk):
    i = pl.program_id(0)

    @pl.when(i == 0)
    def _init():
        m_ref[...] = jnp.full_like(m_ref, -jnp.inf)
        s_ref[...] = jnp.zeros_like(s_ref)
        tl_ref[...] = jnp.zeros_like(tl_ref)

    f = f_ref[...]
    dn = (((1,), (1,)), ((), ()))
    acc = jax.lax.dot_general(f[:, 0, :], l1_ref[...], dn,
                              preferred_element_type=jnp.float32)
    acc += jax.lax.dot_general(f[:, 1, :], l2_ref[...], dn,
                               preferred_element_type=jnp.float32)
    acc += jax.lax.dot_general(f[:, 2, :], l3_ref[...], dn,
                               preferred_element_type=jnp.float32)
    acc += jax.lax.dot_general(f[:, 3, :], l4_ref[...], dn,
                               preferred_element_type=jnp.float32)
    logits_ref[...] = acc

    col = jax.lax.broadcasted_iota(jnp.int32, acc.shape, 1) + i * cblk
    valid = col < nclasses
    masked = jnp.where(valid, acc, -jnp.inf)
    bmax = jnp.max(masked, axis=1, keepdims=True)
    m_old = m_ref[...]
    m_new = jnp.maximum(m_old, bmax)
    p = jnp.where(valid, jnp.exp(acc - m_new), 0.0)
    s_ref[...] = s_ref[...] * jnp.exp(m_old - m_new) + jnp.sum(
        p, axis=1, keepdims=True)
    m_ref[...] = m_new

    t = t_ref[...]
    tl_ref[...] += jnp.sum(jnp.where(col == t, acc, 0.0), axis=1,
                           keepdims=True)

    @pl.when(i == nblk - 1)
    def _fin():
        lse = m_ref[...] + jnp.log(s_ref[...])
        loss_ref[...] = jnp.mean(lse - tl_ref[...]).reshape(1, 1)


def kernel(features, scores, targets, flags, lut_b1, lut_b2, lut_b3,
           lut_b4):
    batch, _, nfeat = features.shape
    nclasses = lut_b1.shape[0]
    nblk = pl.cdiv(nclasses, _C_BLK)
    t2 = targets.astype(jnp.int32).reshape(batch, 1)

    body = functools.partial(_oim_body, nblk=nblk, nclasses=nclasses,
                             cblk=_C_BLK)
    logits, loss = pl.pallas_call(
        body,
        grid=(nblk,),
        in_specs=[
            pl.BlockSpec((batch, 4, nfeat), lambda i: (0, 0, 0)),
            pl.BlockSpec((batch, 1), lambda i: (0, 0)),
            pl.BlockSpec((_C_BLK, nfeat), lambda i: (i, 0)),
            pl.BlockSpec((_C_BLK, nfeat), lambda i: (i, 0)),
            pl.BlockSpec((_C_BLK, nfeat), lambda i: (i, 0)),
            pl.BlockSpec((_C_BLK, nfeat), lambda i: (i, 0)),
        ],
        out_specs=[
            pl.BlockSpec((batch, _C_BLK), lambda i: (0, i)),
            pl.BlockSpec((1, 1), lambda i: (0, 0)),
        ],
        out_shape=[
            jax.ShapeDtypeStruct((batch, nclasses), jnp.float32),
            jax.ShapeDtypeStruct((1, 1), jnp.float32),
        ],
        scratch_shapes=[
            pltpu.VMEM((batch, 1), jnp.float32),
            pltpu.VMEM((batch, 1), jnp.float32),
            pltpu.VMEM((batch, 1), jnp.float32),
        ],
        compiler_params=pltpu.CompilerParams(
            dimension_semantics=("arbitrary",)),
    )(features, t2, lut_b1, lut_b2, lut_b3, lut_b4)
    return (loss[0, 0], logits)


# speedup vs baseline: 2.1090x; 1.0320x over previous
import functools

import jax
import jax.numpy as jnp
from jax.experimental import pallas as pl
from jax.experimental.pallas import tpu as pltpu

_NUM_CLASSES = 100000
_C_BLK = 10240


def _oim_body(f_ref, t_ref, l1_ref, l2_ref, l3_ref, l4_ref,
              logits_ref, loss_ref, m_ref, s_ref, tl_ref,
              *, nblk, nclasses, cblk):
    i = pl.program_id(0)

    @pl.when(i == 0)
    def _init():
        m_ref[...] = jnp.full_like(m_ref, -jnp.inf)
        s_ref[...] = jnp.zeros_like(s_ref)
        tl_ref[...] = jnp.zeros_like(tl_ref)

    f = f_ref[...]
    dn = (((1,), (1,)), ((), ()))
    acc = jax.lax.dot_general(f[:, 0, :], l1_ref[...], dn,
                              preferred_element_type=jnp.float32)
    acc += jax.lax.dot_general(f[:, 1, :], l2_ref[...], dn,
                               preferred_element_type=jnp.float32)
    acc += jax.lax.dot_general(f[:, 2, :], l3_ref[...], dn,
                               preferred_element_type=jnp.float32)
    acc += jax.lax.dot_general(f[:, 3, :], l4_ref[...], dn,
                               preferred_element_type=jnp.float32)
    logits_ref[...] = acc

    col = jax.lax.broadcasted_iota(jnp.int32, acc.shape, 1) + i * cblk
    valid = col < nclasses
    masked = jnp.where(valid, acc, -jnp.inf)
    bmax = jnp.max(masked, axis=1, keepdims=True)
    m_old = m_ref[...]
    m_new = jnp.maximum(m_old, bmax)
    p = jnp.where(valid, jnp.exp(acc - m_new), 0.0)
    s_ref[...] = s_ref[...] * jnp.exp(m_old - m_new) + jnp.sum(
        p, axis=1, keepdims=True)
    m_ref[...] = m_new

    t = t_ref[...]
    tl_ref[...] += jnp.sum(jnp.where(col == t, acc, 0.0), axis=1,
                           keepdims=True)

    @pl.when(i == nblk - 1)
    def _fin():
        lse = m_ref[...] + jnp.log(s_ref[...])
        loss_ref[...] = jnp.mean(lse - tl_ref[...]).reshape(1, 1)


def kernel(features, scores, targets, flags, lut_b1, lut_b2, lut_b3,
           lut_b4):
    batch, _, nfeat = features.shape
    nclasses = lut_b1.shape[0]
    nblk = pl.cdiv(nclasses, _C_BLK)
    t2 = targets.astype(jnp.int32).reshape(batch, 1)

    body = functools.partial(_oim_body, nblk=nblk, nclasses=nclasses,
                             cblk=_C_BLK)
    logits, loss = pl.pallas_call(
        body,
        grid=(nblk,),
        in_specs=[
            pl.BlockSpec((batch, 4, nfeat), lambda i: (0, 0, 0)),
            pl.BlockSpec((batch, 1), lambda i: (0, 0)),
            pl.BlockSpec((_C_BLK, nfeat), lambda i: (i, 0)),
            pl.BlockSpec((_C_BLK, nfeat), lambda i: (i, 0)),
            pl.BlockSpec((_C_BLK, nfeat), lambda i: (i, 0)),
            pl.BlockSpec((_C_BLK, nfeat), lambda i: (i, 0)),
        ],
        out_specs=[
            pl.BlockSpec((batch, _C_BLK), lambda i: (0, i)),
            pl.BlockSpec((1, 1), lambda i: (0, 0)),
        ],
        out_shape=[
            jax.ShapeDtypeStruct((batch, nclasses), jnp.float32),
            jax.ShapeDtypeStruct((1, 1), jnp.float32),
        ],
        scratch_shapes=[
            pltpu.VMEM((batch, 1), jnp.float32),
            pltpu.VMEM((batch, 1), jnp.float32),
            pltpu.VMEM((batch, 1), jnp.float32),
        ],
        compiler_params=pltpu.CompilerParams(
            dimension_semantics=("arbitrary",)),
    )(features, t2, lut_b1, lut_b2, lut_b3, lut_b4)
    return (loss[0, 0], logits)
